# baseline (speedup 1.0000x reference)
import jax
import jax.numpy as jnp
from jax.experimental import pallas as pl
from jax.experimental.pallas import tpu as pltpu


def _router_kernel(x_ref, wg_ref, bg_ref, wmat_ref):
    x = x_ref[...]
    glog = jnp.dot(x, wg_ref[...], preferred_element_type=jnp.float32) + bg_ref[...]
    ii = jax.lax.broadcasted_iota(jnp.int32, glog.shape, 1)
    ne = glog.shape[1]
    m1 = jnp.max(glog, axis=1, keepdims=True)
    i1 = jnp.min(jnp.where(glog >= m1, ii, ne), axis=1, keepdims=True)
    neg = jnp.finfo(jnp.float32).min
    g2 = jnp.where(ii == i1, neg, glog)
    m2 = jnp.max(g2, axis=1, keepdims=True)
    i2 = jnp.min(jnp.where(g2 >= m2, ii, ne), axis=1, keepdims=True)
    p2 = jnp.exp(m2 - m1)
    denom = 1.0 + p2
    wmat_ref[...] = jnp.where(ii == i1, 1.0 / denom,
                              jnp.where(ii == i2, p2 / denom, 0.0))


def _expert_kernel(x_ref, wmat_ref, w1_ref, b1_ref, w2_ref, b2_ref, out_ref):
    e = pl.program_id(1)
    x = x_ref[...]
    wmat = wmat_ref[...]
    ii = jax.lax.broadcasted_iota(jnp.int32, wmat.shape, 1)

    @pl.when(e == 0)
    def _():
        out_ref[...] = jnp.dot(wmat, b2_ref[...],
                               preferred_element_type=jnp.float32)

    we = jnp.sum(jnp.where(ii == e, wmat, 0.0), axis=1, keepdims=True)
    h = jnp.maximum(
        jnp.dot(x, w1_ref[0], preferred_element_type=jnp.float32) + b1_ref[0],
        0.0)
    out_ref[...] += jnp.dot(we * h, w2_ref[0],
                            preferred_element_type=jnp.float32)


def kernel(x, Wg, bg, W1, b1, W2, b2):
    B, D = x.shape
    E = Wg.shape[1]
    wmat = pl.pallas_call(
        _router_kernel,
        grid=(1,),
        in_specs=[
            pl.BlockSpec((B, D), lambda i: (0, 0)),
            pl.BlockSpec((D, E), lambda i: (0, 0)),
            pl.BlockSpec((1, E), lambda i: (0, 0)),
        ],
        out_specs=pl.BlockSpec((B, E), lambda i: (0, 0)),
        out_shape=jax.ShapeDtypeStruct((B, E), jnp.float32),
    )(x, Wg, bg.reshape(1, E))

    TB = B // 2
    out = pl.pallas_call(
        _expert_kernel,
        grid=(2, E),
        in_specs=[
            pl.BlockSpec((TB, D), lambda i, e: (i, 0)),
            pl.BlockSpec((TB, E), lambda i, e: (i, 0)),
            pl.BlockSpec((1, D, D), lambda i, e: (e, 0, 0)),
            pl.BlockSpec((1, 1, D), lambda i, e: (e, 0, 0)),
            pl.BlockSpec((1, D, D), lambda i, e: (e, 0, 0)),
            pl.BlockSpec((E, D), lambda i, e: (0, 0)),
        ],
        out_specs=pl.BlockSpec((TB, D), lambda i, e: (i, 0)),
        out_shape=jax.ShapeDtypeStruct((B, D), jnp.float32),
        compiler_params=pltpu.CompilerParams(
            dimension_semantics=("arbitrary", "arbitrary")),
    )(x, wmat, W1, b1.reshape(E, 1, D), W2, b2)
    return out


# speedup vs baseline: 1.0640x; 1.0640x over previous
import jax
import jax.numpy as jnp
from jax.experimental import pallas as pl
from jax.experimental.pallas import tpu as pltpu

_TB = 512


def _router_kernel(x_ref, wg_ref, bg_ref, wmat_ref):
    x = x_ref[...]
    glog = jnp.dot(x, wg_ref[...], preferred_element_type=jnp.float32) + bg_ref[...]
    ii = jax.lax.broadcasted_iota(jnp.int32, glog.shape, 1)
    ne = glog.shape[1]
    m1 = jnp.max(glog, axis=1, keepdims=True)
    i1 = jnp.min(jnp.where(glog >= m1, ii, ne), axis=1, keepdims=True)
    neg = jnp.finfo(jnp.float32).min
    g2 = jnp.where(ii == i1, neg, glog)
    m2 = jnp.max(g2, axis=1, keepdims=True)
    i2 = jnp.min(jnp.where(g2 >= m2, ii, ne), axis=1, keepdims=True)
    p2 = jnp.exp(m2 - m1)
    denom = 1.0 + p2
    wmat_ref[...] = jnp.where(ii == i1, 1.0 / denom,
                              jnp.where(ii == i2, p2 / denom, 0.0))


def _expert_kernel(x_ref, wmat_ref, w1_ref, b1_ref, w2r_ref, b2_ref, out_ref):
    x = x_ref[...]
    wmat = wmat_ref[...]
    ii = jax.lax.broadcasted_iota(jnp.int32, wmat.shape, 1)
    E = wmat.shape[1]
    hs = []
    for e in range(E):
        we = jnp.sum(jnp.where(ii == e, wmat, 0.0), axis=1, keepdims=True)
        h = jnp.maximum(
            jnp.dot(x, w1_ref[e], preferred_element_type=jnp.float32) + b1_ref[e],
            0.0)
        hs.append(we * h)
    H = jnp.concatenate(hs, axis=1)
    out = jnp.dot(H, w2r_ref[...], preferred_element_type=jnp.float32)
    out += jnp.dot(wmat, b2_ref[...], preferred_element_type=jnp.float32)
    out_ref[...] = out


def kernel(x, Wg, bg, W1, b1, W2, b2):
    B, D = x.shape
    E = Wg.shape[1]
    wmat = pl.pallas_call(
        _router_kernel,
        grid=(1,),
        in_specs=[
            pl.BlockSpec((B, D), lambda i: (0, 0)),
            pl.BlockSpec((D, E), lambda i: (0, 0)),
            pl.BlockSpec((1, E), lambda i: (0, 0)),
        ],
        out_specs=pl.BlockSpec((B, E), lambda i: (0, 0)),
        out_shape=jax.ShapeDtypeStruct((B, E), jnp.float32),
    )(x, Wg, bg.reshape(1, E))

    nb = B // _TB
    out = pl.pallas_call(
        _expert_kernel,
        grid=(nb,),
        in_specs=[
            pl.BlockSpec((_TB, D), lambda i: (i, 0)),
            pl.BlockSpec((_TB, E), lambda i: (i, 0)),
            pl.BlockSpec((E, D, D), lambda i: (0, 0, 0)),
            pl.BlockSpec((E, 1, D), lambda i: (0, 0, 0)),
            pl.BlockSpec((E * D, D), lambda i: (0, 0)),
            pl.BlockSpec((E, D), lambda i: (0, 0)),
        ],
        out_specs=pl.BlockSpec((_TB, D), lambda i: (i, 0)),
        out_shape=jax.ShapeDtypeStruct((B, D), jnp.float32),
        compiler_params=pltpu.CompilerParams(
            dimension_semantics=("parallel",)),
    )(x, wmat, W1, b1.reshape(E, 1, D), W2.reshape(E * D, D), b2)
    return out
